# baseline (device time: 761800 ns/iter reference)
import jax
import jax.numpy as jnp
from jax import lax
from jax.experimental import pallas as pl
from jax.experimental.pallas import tpu as pltpu

N_DEV = 16


def kernel(x, w_mat):
    m_per, k = x.shape
    _, n_per = w_mat.shape
    m_total = N_DEV * m_per

    def body(x_ref, w_ref, out_ref, comm_ref, send_sems, recv_sems,
             credit_sems, amax_src_ref, amax_all_ref, amax_send_sems,
             amax_recv_sems):
        my = lax.axis_index("i")
        left = (my - 1) % N_DEV
        right = (my + 1) % N_DEV

        barrier_sem = pltpu.get_barrier_semaphore()
        for nbr in (left, right):
            pl.semaphore_signal(barrier_sem, inc=1, device_id=(nbr,),
                                device_id_type=pl.DeviceIdType.MESH)
        pl.semaphore_wait(barrier_sem, 2)

        out_ref[pl.ds(my * m_per, m_per), :] = jnp.dot(
            x_ref[...], w_ref[...], preferred_element_type=jnp.float32)

        for h in range(N_DEV - 1):
            send_slot = h % 2
            recv_slot = (h + 1) % 2
            if h >= 2:
                pl.semaphore_wait(credit_sems.at[recv_slot], 1)
            src = x_ref if h == 0 else comm_ref.at[send_slot]
            rdma = pltpu.make_async_remote_copy(
                src_ref=src,
                dst_ref=comm_ref.at[recv_slot],
                send_sem=send_sems.at[send_slot],
                recv_sem=recv_sems.at[recv_slot],
                device_id=(right,),
                device_id_type=pl.DeviceIdType.MESH,
            )
            rdma.start()
            rdma.wait()
            if 1 <= h <= N_DEV - 3:
                pl.semaphore_signal(credit_sems.at[send_slot], inc=1,
                                    device_id=(left,),
                                    device_id_type=pl.DeviceIdType.MESH)
            origin = (my - h - 1) % N_DEV
            out_ref[pl.ds(origin * m_per, m_per), :] = jnp.dot(
                comm_ref[recv_slot], w_ref[...],
                preferred_element_type=jnp.float32)

        y_abs_max = jnp.max(jnp.abs(out_ref[...]))
        amax_src_ref[...] = jnp.full((8, 128), y_abs_max, jnp.float32)

        descs = []
        for d in range(1, N_DEV):
            tgt = (my + d) % N_DEV
            off = N_DEV - d
            rd = pltpu.make_async_remote_copy(
                src_ref=amax_src_ref,
                dst_ref=amax_all_ref.at[off],
                send_sem=amax_send_sems.at[off],
                recv_sem=amax_recv_sems.at[off],
                device_id=(tgt,),
                device_id_type=pl.DeviceIdType.MESH,
            )
            rd.start()
            descs.append(rd)
        for rd in descs:
            rd.wait()

        g = y_abs_max
        for off in range(1, N_DEV):
            g = jnp.maximum(g, amax_all_ref[off, 0, 0])

        scale = g / 127.0
        q = jnp.clip(jnp.round(out_ref[...] / scale), -127.0, 127.0)
        out_ref[...] = q * scale

    return pl.pallas_call(
        body,
        out_shape=jax.ShapeDtypeStruct((m_total, n_per), jnp.float32),
        in_specs=[pl.BlockSpec(memory_space=pltpu.VMEM),
                  pl.BlockSpec(memory_space=pltpu.VMEM)],
        out_specs=pl.BlockSpec(memory_space=pltpu.VMEM),
        scratch_shapes=[
            pltpu.VMEM((2, m_per, k), jnp.float32),
            pltpu.SemaphoreType.DMA((2,)),
            pltpu.SemaphoreType.DMA((2,)),
            pltpu.SemaphoreType.REGULAR((2,)),
            pltpu.VMEM((8, 128), jnp.float32),
            pltpu.VMEM((N_DEV, 8, 128), jnp.float32),
            pltpu.SemaphoreType.DMA((N_DEV,)),
            pltpu.SemaphoreType.DMA((N_DEV,)),
        ],
        compiler_params=pltpu.CompilerParams(collective_id=0),
    )(x, w_mat)


# device time: 411234 ns/iter; 1.8525x vs baseline; 1.8525x over previous
import jax
import jax.numpy as jnp
from jax import lax
from jax.experimental import pallas as pl
from jax.experimental.pallas import tpu as pltpu

N_DEV = 16
CW_HOPS = 8
CCW_HOPS = 7


def kernel(x, w_mat):
    m_per, k = x.shape
    _, n_per = w_mat.shape
    m_total = N_DEV * m_per

    def body(x_ref, w_ref, out_ref, cw_ref, ccw_ref,
             cw_send_sems, cw_recv_sems, ccw_send_sems, ccw_recv_sems,
             cw_credit, ccw_credit, amax_src_ref, amax_all_ref,
             amax_send_sems, amax_recv_sems):
        my = lax.axis_index("i")
        left = (my - 1) % N_DEV
        right = (my + 1) % N_DEV

        barrier_sem = pltpu.get_barrier_semaphore()
        for nbr in (left, right):
            pl.semaphore_signal(barrier_sem, inc=1, device_id=(nbr,),
                                device_id_type=pl.DeviceIdType.MESH)
        pl.semaphore_wait(barrier_sem, 2)

        def gemm_store(chunk, origin):
            blk = jnp.dot(chunk, w_ref[...],
                          preferred_element_type=jnp.float32)
            out_ref[pl.ds(origin * m_per, m_per), :] = blk
            return jnp.max(jnp.abs(blk))

        amax_acc = jnp.float32(0.0)

        for h in range(CW_HOPS):
            s = h % 2
            r = (h + 1) % 2
            if h >= 2:
                pl.semaphore_wait(cw_credit.at[r], 1)
            if 2 <= h < CCW_HOPS:
                pl.semaphore_wait(ccw_credit.at[r], 1)
            cw = pltpu.make_async_remote_copy(
                src_ref=x_ref if h == 0 else cw_ref.at[s],
                dst_ref=cw_ref.at[r],
                send_sem=cw_send_sems.at[s],
                recv_sem=cw_recv_sems.at[r],
                device_id=(right,),
                device_id_type=pl.DeviceIdType.MESH,
            )
            cw.start()
            ccw = None
            if h < CCW_HOPS:
                ccw = pltpu.make_async_remote_copy(
                    src_ref=x_ref if h == 0 else ccw_ref.at[s],
                    dst_ref=ccw_ref.at[r],
                    send_sem=ccw_send_sems.at[s],
                    recv_sem=ccw_recv_sems.at[r],
                    device_id=(left,),
                    device_id_type=pl.DeviceIdType.MESH,
                )
                ccw.start()
            if h == 0:
                amax_acc = jnp.maximum(amax_acc,
                                       gemm_store(x_ref[...], my))
            else:
                amax_acc = jnp.maximum(
                    amax_acc, gemm_store(cw_ref[s], (my - h) % N_DEV))
                amax_acc = jnp.maximum(
                    amax_acc, gemm_store(ccw_ref[s], (my + h) % N_DEV))
            cw.wait()
            if ccw is not None:
                ccw.wait()
            if 1 <= h <= CW_HOPS - 2:
                pl.semaphore_signal(cw_credit.at[s], inc=1,
                                    device_id=(left,),
                                    device_id_type=pl.DeviceIdType.MESH)
            if 1 <= h <= CCW_HOPS - 2:
                pl.semaphore_signal(ccw_credit.at[s], inc=1,
                                    device_id=(right,),
                                    device_id_type=pl.DeviceIdType.MESH)

        amax_acc = jnp.maximum(
            amax_acc, gemm_store(cw_ref[CW_HOPS % 2],
                                 (my - CW_HOPS) % N_DEV))
        amax_acc = jnp.maximum(
            amax_acc, gemm_store(ccw_ref[CCW_HOPS % 2],
                                 (my + CCW_HOPS) % N_DEV))

        amax_src_ref[...] = jnp.full((8, 128), amax_acc, jnp.float32)
        descs = []
        for d in range(1, N_DEV):
            tgt = (my + d) % N_DEV
            off = N_DEV - d
            rd = pltpu.make_async_remote_copy(
                src_ref=amax_src_ref,
                dst_ref=amax_all_ref.at[off],
                send_sem=amax_send_sems.at[off],
                recv_sem=amax_recv_sems.at[off],
                device_id=(tgt,),
                device_id_type=pl.DeviceIdType.MESH,
            )
            rd.start()
            descs.append(rd)
        for rd in descs:
            rd.wait()

        g = amax_acc
        for off in range(1, N_DEV):
            g = jnp.maximum(g, amax_all_ref[off, 0, 0])

        scale = g / 127.0
        q = jnp.clip(jnp.round(out_ref[...] / scale), -127.0, 127.0)
        out_ref[...] = q * scale

    return pl.pallas_call(
        body,
        out_shape=jax.ShapeDtypeStruct((m_total, n_per), jnp.float32),
        in_specs=[pl.BlockSpec(memory_space=pltpu.VMEM),
                  pl.BlockSpec(memory_space=pltpu.VMEM)],
        out_specs=pl.BlockSpec(memory_space=pltpu.VMEM),
        scratch_shapes=[
            pltpu.VMEM((2, m_per, k), jnp.float32),
            pltpu.VMEM((2, m_per, k), jnp.float32),
            pltpu.SemaphoreType.DMA((2,)),
            pltpu.SemaphoreType.DMA((2,)),
            pltpu.SemaphoreType.DMA((2,)),
            pltpu.SemaphoreType.DMA((2,)),
            pltpu.SemaphoreType.REGULAR((2,)),
            pltpu.SemaphoreType.REGULAR((2,)),
            pltpu.VMEM((8, 128), jnp.float32),
            pltpu.VMEM((N_DEV, 8, 128), jnp.float32),
            pltpu.SemaphoreType.DMA((N_DEV,)),
            pltpu.SemaphoreType.DMA((N_DEV,)),
        ],
        compiler_params=pltpu.CompilerParams(collective_id=0),
    )(x, w_mat)


# device time: 389683 ns/iter; 1.9549x vs baseline; 1.0553x over previous
import jax
import jax.numpy as jnp
from jax import lax
from jax.experimental import pallas as pl
from jax.experimental.pallas import tpu as pltpu

N_DEV = 16
HOPS = 8


def kernel(x, w_mat):
    m_per, k = x.shape
    _, n_per = w_mat.shape
    m_total = N_DEV * m_per

    def body(x_ref, w_ref, out_ref, cw_ref, ccw_ref,
             cw_send_sems, cw_recv_sems, ccw_send_sems, ccw_recv_sems,
             cw_credit, ccw_credit, amax_src_ref, amax_all_ref,
             amax_send_sems, amax_recv_sems):
        my = lax.axis_index("i")
        left = (my - 1) % N_DEV
        right = (my + 1) % N_DEV

        barrier_sem = pltpu.get_barrier_semaphore()
        for nbr in (left, right):
            pl.semaphore_signal(barrier_sem, inc=1, device_id=(nbr,),
                                device_id_type=pl.DeviceIdType.MESH)
        pl.semaphore_wait(barrier_sem, 2)

        def gemm_store(chunk, origin):
            blk = jnp.dot(chunk, w_ref[...],
                          preferred_element_type=jnp.float32)
            out_ref[pl.ds(origin * m_per, m_per), :] = blk
            return jnp.max(jnp.abs(blk))

        amax_acc = jnp.float32(0.0)

        half = m_per // 2
        for h in range(HOPS):
            s = h % 2
            r = (h + 1) % 2
            if h >= 2:
                pl.semaphore_wait(cw_credit.at[r], 1)
                pl.semaphore_wait(ccw_credit.at[r], 1)
            last = h == HOPS - 1
            if h == 0:
                cw_src, cw_dst = x_ref, cw_ref.at[r]
                ccw_src, ccw_dst = x_ref, ccw_ref.at[r]
            elif last:
                cw_src = cw_ref.at[s, pl.ds(0, half)]
                cw_dst = cw_ref.at[r, pl.ds(0, half)]
                ccw_src = ccw_ref.at[s, pl.ds(half, half)]
                ccw_dst = ccw_ref.at[r, pl.ds(half, half)]
            else:
                cw_src, cw_dst = cw_ref.at[s], cw_ref.at[r]
                ccw_src, ccw_dst = ccw_ref.at[s], ccw_ref.at[r]
            cw = pltpu.make_async_remote_copy(
                src_ref=cw_src,
                dst_ref=cw_dst,
                send_sem=cw_send_sems.at[s],
                recv_sem=cw_recv_sems.at[r],
                device_id=(right,),
                device_id_type=pl.DeviceIdType.MESH,
            )
            cw.start()
            ccw = pltpu.make_async_remote_copy(
                src_ref=ccw_src,
                dst_ref=ccw_dst,
                send_sem=ccw_send_sems.at[s],
                recv_sem=ccw_recv_sems.at[r],
                device_id=(left,),
                device_id_type=pl.DeviceIdType.MESH,
            )
            ccw.start()
            if h == 0:
                amax_acc = jnp.maximum(amax_acc,
                                       gemm_store(x_ref[...], my))
            else:
                amax_acc = jnp.maximum(
                    amax_acc, gemm_store(cw_ref[s], (my - h) % N_DEV))
                amax_acc = jnp.maximum(
                    amax_acc, gemm_store(ccw_ref[s], (my + h) % N_DEV))
            cw.wait()
            ccw.wait()
            if 1 <= h <= HOPS - 2:
                pl.semaphore_signal(cw_credit.at[s], inc=1,
                                    device_id=(left,),
                                    device_id_type=pl.DeviceIdType.MESH)
                pl.semaphore_signal(ccw_credit.at[s], inc=1,
                                    device_id=(right,),
                                    device_id_type=pl.DeviceIdType.MESH)

        far = (my + HOPS) % N_DEV
        blk_top = jnp.dot(cw_ref[HOPS % 2, :half, :], w_ref[...],
                          preferred_element_type=jnp.float32)
        out_ref[pl.ds(far * m_per, half), :] = blk_top
        blk_bot = jnp.dot(ccw_ref[HOPS % 2, half:, :], w_ref[...],
                          preferred_element_type=jnp.float32)
        out_ref[pl.ds(far * m_per + half, half), :] = blk_bot
        amax_acc = jnp.maximum(amax_acc, jnp.max(jnp.abs(blk_top)))
        amax_acc = jnp.maximum(amax_acc, jnp.max(jnp.abs(blk_bot)))

        amax_src_ref[...] = jnp.full((8, 128), amax_acc, jnp.float32)
        descs = []
        for d in range(1, N_DEV):
            tgt = (my + d) % N_DEV
            off = N_DEV - d
            rd = pltpu.make_async_remote_copy(
                src_ref=amax_src_ref,
                dst_ref=amax_all_ref.at[off],
                send_sem=amax_send_sems.at[off],
                recv_sem=amax_recv_sems.at[off],
                device_id=(tgt,),
                device_id_type=pl.DeviceIdType.MESH,
            )
            rd.start()
            descs.append(rd)
        for rd in descs:
            rd.wait()

        g = amax_acc
        for off in range(1, N_DEV):
            g = jnp.maximum(g, amax_all_ref[off, 0, 0])

        scale = g / 127.0
        q = jnp.clip(jnp.round(out_ref[...] / scale), -127.0, 127.0)
        out_ref[...] = q * scale

    return pl.pallas_call(
        body,
        out_shape=jax.ShapeDtypeStruct((m_total, n_per), jnp.float32),
        in_specs=[pl.BlockSpec(memory_space=pltpu.VMEM),
                  pl.BlockSpec(memory_space=pltpu.VMEM)],
        out_specs=pl.BlockSpec(memory_space=pltpu.VMEM),
        scratch_shapes=[
            pltpu.VMEM((2, m_per, k), jnp.float32),
            pltpu.VMEM((2, m_per, k), jnp.float32),
            pltpu.SemaphoreType.DMA((2,)),
            pltpu.SemaphoreType.DMA((2,)),
            pltpu.SemaphoreType.DMA((2,)),
            pltpu.SemaphoreType.DMA((2,)),
            pltpu.SemaphoreType.REGULAR((2,)),
            pltpu.SemaphoreType.REGULAR((2,)),
            pltpu.VMEM((8, 128), jnp.float32),
            pltpu.VMEM((N_DEV, 8, 128), jnp.float32),
            pltpu.SemaphoreType.DMA((N_DEV,)),
            pltpu.SemaphoreType.DMA((N_DEV,)),
        ],
        compiler_params=pltpu.CompilerParams(collective_id=0),
    )(x, w_mat)


# device time: 377232 ns/iter; 2.0194x vs baseline; 1.0330x over previous
import jax
import jax.numpy as jnp
from jax import lax
from jax.experimental import pallas as pl
from jax.experimental.pallas import tpu as pltpu

N_DEV = 16
HOPS = 8


def kernel(x, w_mat):
    m_per, k = x.shape
    _, n_per = w_mat.shape
    m_total = N_DEV * m_per

    def body(x_ref, w_ref, out_ref, cw_ref, ccw_ref,
             cw_send_sems, cw_recv_sems, ccw_send_sems, ccw_recv_sems,
             cw_credit, ccw_credit, amax_src_ref, amax_all_ref,
             amax_send_sems, amax_recv_sems):
        my = lax.axis_index("i")
        left = (my - 1) % N_DEV
        right = (my + 1) % N_DEV

        barrier_sem = pltpu.get_barrier_semaphore()
        for nbr in (left, right):
            pl.semaphore_signal(barrier_sem, inc=1, device_id=(nbr,),
                                device_id_type=pl.DeviceIdType.MESH)
        pl.semaphore_wait(barrier_sem, 2)

        def gemm_store(chunk, origin):
            blk = jnp.dot(chunk, w_ref[...],
                          preferred_element_type=jnp.float32)
            out_ref[pl.ds(origin * m_per, m_per), :] = blk
            return jnp.max(jnp.abs(blk))

        amax_acc = jnp.float32(0.0)

        half = m_per // 2
        for h in range(HOPS):
            s = h % 3
            r = (h + 1) % 3
            if h >= 3:
                pl.semaphore_wait(cw_credit.at[r], 1)
                pl.semaphore_wait(ccw_credit.at[r], 1)
            last = h == HOPS - 1
            if h == 0:
                cw_src, cw_dst = x_ref, cw_ref.at[r]
                ccw_src, ccw_dst = x_ref, ccw_ref.at[r]
            elif last:
                cw_src = cw_ref.at[s, pl.ds(0, half)]
                cw_dst = cw_ref.at[r, pl.ds(0, half)]
                ccw_src = ccw_ref.at[s, pl.ds(half, half)]
                ccw_dst = ccw_ref.at[r, pl.ds(half, half)]
            else:
                cw_src, cw_dst = cw_ref.at[s], cw_ref.at[r]
                ccw_src, ccw_dst = ccw_ref.at[s], ccw_ref.at[r]
            cw = pltpu.make_async_remote_copy(
                src_ref=cw_src,
                dst_ref=cw_dst,
                send_sem=cw_send_sems.at[s],
                recv_sem=cw_recv_sems.at[r],
                device_id=(right,),
                device_id_type=pl.DeviceIdType.MESH,
            )
            cw.start()
            ccw = pltpu.make_async_remote_copy(
                src_ref=ccw_src,
                dst_ref=ccw_dst,
                send_sem=ccw_send_sems.at[s],
                recv_sem=ccw_recv_sems.at[r],
                device_id=(left,),
                device_id_type=pl.DeviceIdType.MESH,
            )
            ccw.start()
            if h == 0:
                amax_acc = jnp.maximum(amax_acc,
                                       gemm_store(x_ref[...], my))
            else:
                amax_acc = jnp.maximum(
                    amax_acc, gemm_store(cw_ref[s], (my - h) % N_DEV))
                amax_acc = jnp.maximum(
                    amax_acc, gemm_store(ccw_ref[s], (my + h) % N_DEV))
            cw.wait()
            ccw.wait()
            if 1 <= h <= HOPS - 3:
                pl.semaphore_signal(cw_credit.at[s], inc=1,
                                    device_id=(left,),
                                    device_id_type=pl.DeviceIdType.MESH)
                pl.semaphore_signal(ccw_credit.at[s], inc=1,
                                    device_id=(right,),
                                    device_id_type=pl.DeviceIdType.MESH)

        far = (my + HOPS) % N_DEV
        tail_slot = HOPS % 3
        blk_top = jnp.dot(cw_ref[tail_slot, :half, :], w_ref[...],
                          preferred_element_type=jnp.float32)
        out_ref[pl.ds(far * m_per, half), :] = blk_top
        blk_bot = jnp.dot(ccw_ref[tail_slot, half:, :], w_ref[...],
                          preferred_element_type=jnp.float32)
        out_ref[pl.ds(far * m_per + half, half), :] = blk_bot
        amax_acc = jnp.maximum(amax_acc, jnp.max(jnp.abs(blk_top)))
        amax_acc = jnp.maximum(amax_acc, jnp.max(jnp.abs(blk_bot)))

        amax_src_ref[...] = jnp.full((8, 128), amax_acc, jnp.float32)
        descs = []
        for d in range(1, N_DEV):
            tgt = (my + d) % N_DEV
            off = N_DEV - d
            rd = pltpu.make_async_remote_copy(
                src_ref=amax_src_ref,
                dst_ref=amax_all_ref.at[off],
                send_sem=amax_send_sems.at[off],
                recv_sem=amax_recv_sems.at[off],
                device_id=(tgt,),
                device_id_type=pl.DeviceIdType.MESH,
            )
            rd.start()
            descs.append(rd)
        for rd in descs:
            rd.wait()

        g = amax_acc
        for off in range(1, N_DEV):
            g = jnp.maximum(g, amax_all_ref[off, 0, 0])

        scale = g / 127.0
        q = jnp.clip(jnp.round(out_ref[...] / scale), -127.0, 127.0)
        out_ref[...] = q * scale

    return pl.pallas_call(
        body,
        out_shape=jax.ShapeDtypeStruct((m_total, n_per), jnp.float32),
        in_specs=[pl.BlockSpec(memory_space=pltpu.VMEM),
                  pl.BlockSpec(memory_space=pltpu.VMEM)],
        out_specs=pl.BlockSpec(memory_space=pltpu.VMEM),
        scratch_shapes=[
            pltpu.VMEM((3, m_per, k), jnp.float32),
            pltpu.VMEM((3, m_per, k), jnp.float32),
            pltpu.SemaphoreType.DMA((3,)),
            pltpu.SemaphoreType.DMA((3,)),
            pltpu.SemaphoreType.DMA((3,)),
            pltpu.SemaphoreType.DMA((3,)),
            pltpu.SemaphoreType.REGULAR((3,)),
            pltpu.SemaphoreType.REGULAR((3,)),
            pltpu.VMEM((8, 128), jnp.float32),
            pltpu.VMEM((N_DEV, 8, 128), jnp.float32),
            pltpu.SemaphoreType.DMA((N_DEV,)),
            pltpu.SemaphoreType.DMA((N_DEV,)),
        ],
        compiler_params=pltpu.CompilerParams(collective_id=0),
    )(x, w_mat)


# device time: 358372 ns/iter; 2.1257x vs baseline; 1.0526x over previous
import jax
import jax.numpy as jnp
from jax import lax
from jax.experimental import pallas as pl
from jax.experimental.pallas import tpu as pltpu

N_DEV = 16
HOPS = 8


def kernel(x, w_mat):
    m_per, k = x.shape
    _, n_per = w_mat.shape
    m_total = N_DEV * m_per
    half = m_per // 2

    def body(x_ref, w_ref, out_ref, cw_ref, ccw_ref,
             cw_send_sems, cw_recv_sems, ccw_send_sems, ccw_recv_sems,
             cw_credit, ccw_credit, amax_src_ref, amax_all_ref,
             amax_send_sems, amax_recv_sems):
        my = lax.axis_index("i")
        left = (my - 1) % N_DEV
        right = (my + 1) % N_DEV

        barrier_sem = pltpu.get_barrier_semaphore()
        for nbr in (left, right):
            pl.semaphore_signal(barrier_sem, inc=1, device_id=(nbr,),
                                device_id_type=pl.DeviceIdType.MESH)
        pl.semaphore_wait(barrier_sem, 2)

        def cw_exists(h, j):
            return h < HOPS - 1 or j == 0

        def ccw_exists(h, j):
            return h < HOPS - 1 or j == 1

        def start_pair(h, j):
            s, r = h % 3, (h + 1) % 3
            row = pl.ds(j * half, half)
            cw_d = ccw_d = None
            if cw_exists(h, j):
                cw_d = pltpu.make_async_remote_copy(
                    src_ref=x_ref.at[row] if h == 0 else cw_ref.at[s, row],
                    dst_ref=cw_ref.at[r, row],
                    send_sem=cw_send_sems.at[s, j],
                    recv_sem=cw_recv_sems.at[r, j],
                    device_id=(right,),
                    device_id_type=pl.DeviceIdType.MESH,
                )
                cw_d.start()
            if ccw_exists(h, j):
                ccw_d = pltpu.make_async_remote_copy(
                    src_ref=x_ref.at[row] if h == 0 else ccw_ref.at[s, row],
                    dst_ref=ccw_ref.at[r, row],
                    send_sem=ccw_send_sems.at[s, j],
                    recv_sem=ccw_recv_sems.at[r, j],
                    device_id=(left,),
                    device_id_type=pl.DeviceIdType.MESH,
                )
                ccw_d.start()
            return cw_d, ccw_d

        def gemm_store(chunk, origin):
            blk = jnp.dot(chunk, w_ref[...],
                          preferred_element_type=jnp.float32)
            out_ref[pl.ds(origin * m_per, m_per), :] = blk
            return jnp.max(jnp.abs(blk))

        prev = [start_pair(0, 0), start_pair(0, 1)]
        amax_acc = gemm_store(x_ref[...], my)

        for h in range(1, HOPS):
            s, r = h % 3, (h + 1) % 3
            cur = []
            for j in (0, 1):
                pc, pcc = prev[j]
                if pc is not None:
                    pc.wait_recv()
                if pcc is not None:
                    pcc.wait_recv()
                if h >= 3:
                    if cw_exists(h, j):
                        pl.semaphore_wait(cw_credit.at[r, j], 1)
                    if ccw_exists(h, j):
                        pl.semaphore_wait(ccw_credit.at[r, j], 1)
                cur.append(start_pair(h, j))
            amax_acc = jnp.maximum(
                amax_acc, gemm_store(cw_ref[s], (my - h) % N_DEV))
            amax_acc = jnp.maximum(
                amax_acc, gemm_store(ccw_ref[s], (my + h) % N_DEV))
            hp = h - 1
            ps = hp % 3
            for j in (0, 1):
                pc, pcc = prev[j]
                if pc is not None:
                    pc.wait_send()
                    if 1 <= hp and cw_exists(hp + 2, j) and hp + 2 <= HOPS - 1:
                        pl.semaphore_signal(
                            cw_credit.at[ps, j], inc=1, device_id=(left,),
                            device_id_type=pl.DeviceIdType.MESH)
                if pcc is not None:
                    pcc.wait_send()
                    if 1 <= hp and ccw_exists(hp + 2, j) and hp + 2 <= HOPS - 1:
                        pl.semaphore_signal(
                            ccw_credit.at[ps, j], inc=1, device_id=(right,),
                            device_id_type=pl.DeviceIdType.MESH)
            prev = cur

        far = (my + HOPS) % N_DEV
        tail_slot = HOPS % 3
        cw_last = prev[0][0]
        ccw_last = prev[1][1]
        cw_last.wait_recv()
        blk_top = jnp.dot(cw_ref[tail_slot, :half, :], w_ref[...],
                          preferred_element_type=jnp.float32)
        out_ref[pl.ds(far * m_per, half), :] = blk_top
        ccw_last.wait_recv()
        blk_bot = jnp.dot(ccw_ref[tail_slot, half:, :], w_ref[...],
                          preferred_element_type=jnp.float32)
        out_ref[pl.ds(far * m_per + half, half), :] = blk_bot
        amax_acc = jnp.maximum(amax_acc, jnp.max(jnp.abs(blk_top)))
        amax_acc = jnp.maximum(amax_acc, jnp.max(jnp.abs(blk_bot)))
        cw_last.wait_send()
        ccw_last.wait_send()

        amax_src_ref[...] = jnp.full((8, 128), amax_acc, jnp.float32)
        descs = []
        for d in range(1, N_DEV):
            tgt = (my + d) % N_DEV
            off = N_DEV - d
            rd = pltpu.make_async_remote_copy(
                src_ref=amax_src_ref,
                dst_ref=amax_all_ref.at[off],
                send_sem=amax_send_sems.at[off],
                recv_sem=amax_recv_sems.at[off],
                device_id=(tgt,),
                device_id_type=pl.DeviceIdType.MESH,
            )
            rd.start()
            descs.append(rd)
        for rd in descs:
            rd.wait()

        g = amax_acc
        for off in range(1, N_DEV):
            g = jnp.maximum(g, amax_all_ref[off, 0, 0])

        inv = 127.0 / g
        q = jnp.clip(jnp.round(out_ref[...] * inv), -127.0, 127.0)
        out_ref[...] = q * (g / 127.0)

    return pl.pallas_call(
        body,
        out_shape=jax.ShapeDtypeStruct((m_total, n_per), jnp.float32),
        in_specs=[pl.BlockSpec(memory_space=pltpu.VMEM),
                  pl.BlockSpec(memory_space=pltpu.VMEM)],
        out_specs=pl.BlockSpec(memory_space=pltpu.VMEM),
        scratch_shapes=[
            pltpu.VMEM((3, m_per, k), jnp.float32),
            pltpu.VMEM((3, m_per, k), jnp.float32),
            pltpu.SemaphoreType.DMA((3, 2)),
            pltpu.SemaphoreType.DMA((3, 2)),
            pltpu.SemaphoreType.DMA((3, 2)),
            pltpu.SemaphoreType.DMA((3, 2)),
            pltpu.SemaphoreType.REGULAR((3, 2)),
            pltpu.SemaphoreType.REGULAR((3, 2)),
            pltpu.VMEM((8, 128), jnp.float32),
            pltpu.VMEM((N_DEV, 8, 128), jnp.float32),
            pltpu.SemaphoreType.DMA((N_DEV,)),
            pltpu.SemaphoreType.DMA((N_DEV,)),
        ],
        compiler_params=pltpu.CompilerParams(collective_id=0),
    )(x, w_mat)
